# baseline (speedup 1.0000x reference)
import functools

import jax
import jax.numpy as jnp
from jax.experimental import pallas as pl
from jax.experimental.pallas import tpu as pltpu

_CONTEXT = 20
_VOCAB = 100000
_EMBED = 64
_HIDDEN = 100

_VB = 4096
_NVB = (_VOCAB + _VB - 1) // _VB


def _hidden_body(idx_ref, emb_ref, w1_ref, b1_ref, h_ref):
    i = pl.program_id(0)
    part = jnp.dot(
        emb_ref[0], w1_ref[...], preferred_element_type=jnp.float32
    )

    @pl.when(i == 0)
    def _():
        h_ref[...] = b1_ref[...] + part

    @pl.when(i > 0)
    def _():
        h_ref[...] += part

    @pl.when(i == _CONTEXT - 1)
    def _():
        h_ref[...] = jnp.tanh(h_ref[...])


def _logits_body(h_ref, w2_ref, b2_ref, logits_ref, lse_ref, stat_ref):
    j = pl.program_id(0)
    x = jnp.dot(h_ref[...], w2_ref[...], preferred_element_type=jnp.float32)
    x = x + b2_ref[...]
    col = j * _VB + jax.lax.broadcasted_iota(jnp.int32, (1, _VB), 1)
    x = jnp.where(col < _VOCAB, x, -jnp.inf)
    logits_ref[...] = x
    bm = jnp.max(x)

    @pl.when(j == 0)
    def _():
        stat_ref[0] = bm
        stat_ref[1] = jnp.sum(jnp.exp(x - bm))

    @pl.when(j > 0)
    def _():
        m_old = stat_ref[0]
        m_new = jnp.maximum(m_old, bm)
        stat_ref[1] = stat_ref[1] * jnp.exp(m_old - m_new) + jnp.sum(
            jnp.exp(x - m_new)
        )
        stat_ref[0] = m_new

    @pl.when(j == _NVB - 1)
    def _():
        lse_ref[0, 0] = stat_ref[0] + jnp.log(stat_ref[1])


def _norm_body(logits_ref, lse_ref, out_ref):
    out_ref[...] = logits_ref[...] - lse_ref[0, 0]


def kernel(inputs, emb_table, W1, b1, W2, b2):
    b1_2d = b1.reshape(1, _HIDDEN)
    b2_2d = b2.reshape(1, _VOCAB)

    h = pl.pallas_call(
        _hidden_body,
        grid_spec=pltpu.PrefetchScalarGridSpec(
            num_scalar_prefetch=1,
            grid=(_CONTEXT,),
            in_specs=[
                pl.BlockSpec((1, 1, _EMBED), lambda i, idx: (idx[i], 0, 0)),
                pl.BlockSpec((_EMBED, _HIDDEN), lambda i, idx: (i, 0)),
                pl.BlockSpec((1, _HIDDEN), lambda i, idx: (0, 0)),
            ],
            out_specs=pl.BlockSpec((1, _HIDDEN), lambda i, idx: (0, 0)),
        ),
        out_shape=jax.ShapeDtypeStruct((1, _HIDDEN), jnp.float32),
    )(
        inputs.astype(jnp.int32),
        emb_table.reshape(_VOCAB, 1, _EMBED),
        W1,
        b1_2d,
    )

    logits, lse = pl.pallas_call(
        _logits_body,
        grid=(_NVB,),
        in_specs=[
            pl.BlockSpec((1, _HIDDEN), lambda j: (0, 0)),
            pl.BlockSpec((_HIDDEN, _VB), lambda j: (0, j)),
            pl.BlockSpec((1, _VB), lambda j: (0, j)),
        ],
        out_specs=[
            pl.BlockSpec((1, _VB), lambda j: (0, j)),
            pl.BlockSpec(memory_space=pltpu.SMEM),
        ],
        out_shape=[
            jax.ShapeDtypeStruct((1, _VOCAB), jnp.float32),
            jax.ShapeDtypeStruct((1, 1), jnp.float32),
        ],
        scratch_shapes=[pltpu.SMEM((2,), jnp.float32)],
    )(h, W2, b2_2d)

    out = pl.pallas_call(
        _norm_body,
        grid=(_NVB,),
        in_specs=[
            pl.BlockSpec((1, _VB), lambda j: (0, j)),
            pl.BlockSpec(memory_space=pltpu.SMEM),
        ],
        out_specs=pl.BlockSpec((1, _VB), lambda j: (0, j)),
        out_shape=jax.ShapeDtypeStruct((1, _VOCAB), jnp.float32),
    )(logits, lse)

    return out


# speedup vs baseline: 1.0778x; 1.0778x over previous
import jax
import jax.numpy as jnp
from jax.experimental import pallas as pl
from jax.experimental.pallas import tpu as pltpu

_CONTEXT = 20
_VOCAB = 100000
_EMBED = 64
_HIDDEN = 100

_VB = 4096
_NVB = (_VOCAB + _VB - 1) // _VB


def _body(
    idx_ref,
    emb_hbm,
    w1_ref,
    b1_ref,
    w2_ref,
    b2_ref,
    logits_in_ref,
    out_ref,
    emb_vmem,
    h_ref,
    stat_ref,
    dma_sem,
):
    p = pl.program_id(0)
    j = pl.program_id(1)

    @pl.when(jnp.logical_and(p == 0, j == 0))
    def _gather_and_hidden():
        for i in range(_CONTEXT):
            pltpu.make_async_copy(
                emb_hbm.at[pl.ds(idx_ref[i], 1), :],
                emb_vmem.at[pl.ds(i, 1), :],
                dma_sem,
            ).start()
        for i in range(_CONTEXT):
            pltpu.make_async_copy(
                emb_hbm.at[pl.ds(idx_ref[i], 1), :],
                emb_vmem.at[pl.ds(i, 1), :],
                dma_sem,
            ).wait()
        acc = b1_ref[...]
        for i in range(_CONTEXT):
            acc = acc + jnp.dot(
                emb_vmem[pl.ds(i, 1), :],
                w1_ref[pl.ds(i * _EMBED, _EMBED), :],
                preferred_element_type=jnp.float32,
            )
        h_ref[...] = jnp.tanh(acc)

    @pl.when(p == 0)
    def _logits_and_stats():
        x = jnp.dot(h_ref[...], w2_ref[...], preferred_element_type=jnp.float32)
        x = x + b2_ref[...]
        col = j * _VB + jax.lax.broadcasted_iota(jnp.int32, (1, _VB), 1)
        x = jnp.where(col < _VOCAB, x, -jnp.inf)
        out_ref[...] = x
        bm = jnp.max(x)

        @pl.when(j == 0)
        def _():
            stat_ref[0] = bm
            stat_ref[1] = jnp.sum(jnp.exp(x - bm))

        @pl.when(j > 0)
        def _():
            m_old = stat_ref[0]
            m_new = jnp.maximum(m_old, bm)
            stat_ref[1] = stat_ref[1] * jnp.exp(m_old - m_new) + jnp.sum(
                jnp.exp(x - m_new)
            )
            stat_ref[0] = m_new

    @pl.when(p == 1)
    def _normalize():
        lse = stat_ref[0] + jnp.log(stat_ref[1])
        out_ref[...] = logits_in_ref[...] - lse


def kernel(inputs, emb_table, W1, b1, W2, b2):
    b1_2d = b1.reshape(1, _HIDDEN)
    b2_2d = b2.reshape(1, _VOCAB)
    logits_buf = jnp.zeros((1, _VOCAB), jnp.float32)

    out = pl.pallas_call(
        _body,
        grid_spec=pltpu.PrefetchScalarGridSpec(
            num_scalar_prefetch=1,
            grid=(2, _NVB),
            in_specs=[
                pl.BlockSpec(memory_space=pl.ANY),
                pl.BlockSpec((_CONTEXT * _EMBED, _HIDDEN), lambda p, j, idx: (0, 0)),
                pl.BlockSpec((1, _HIDDEN), lambda p, j, idx: (0, 0)),
                pl.BlockSpec(
                    (_HIDDEN, _VB),
                    lambda p, j, idx: (0, jax.lax.select(p == 0, j, 0)),
                ),
                pl.BlockSpec(
                    (1, _VB),
                    lambda p, j, idx: (0, jax.lax.select(p == 0, j, 0)),
                ),
                pl.BlockSpec((1, _VB), lambda p, j, idx: (0, j)),
            ],
            out_specs=pl.BlockSpec((1, _VB), lambda p, j, idx: (0, j)),
            scratch_shapes=[
                pltpu.VMEM((_CONTEXT, _EMBED), jnp.float32),
                pltpu.VMEM((1, _HIDDEN), jnp.float32),
                pltpu.SMEM((2,), jnp.float32),
                pltpu.SemaphoreType.DMA,
            ],
        ),
        out_shape=jax.ShapeDtypeStruct((1, _VOCAB), jnp.float32),
        input_output_aliases={6: 0},
        compiler_params=pltpu.CompilerParams(
            dimension_semantics=("arbitrary", "arbitrary"),
        ),
    )(inputs.astype(jnp.int32), emb_table, W1, b1_2d, W2, b2_2d, logits_buf)

    return out


# speedup vs baseline: 1.2907x; 1.1976x over previous
import jax
import jax.numpy as jnp
from jax.experimental import pallas as pl
from jax.experimental.pallas import tpu as pltpu

_CONTEXT = 20
_VOCAB = 100000
_EMBED = 64
_HIDDEN = 100

_VB = 8192
_NVB = (_VOCAB + _VB - 1) // _VB


def _body(
    idx_ref,
    emb_hbm,
    w1_ref,
    b1_ref,
    w2_ref,
    b2_ref,
    logits_in_ref,
    out_ref,
    emb_vmem,
    h_ref,
    stat_ref,
    dma_sem,
):
    p = pl.program_id(0)
    j = pl.program_id(1)

    @pl.when(jnp.logical_and(p == 0, j == 0))
    def _gather_and_hidden():
        for i in range(_CONTEXT):
            pltpu.make_async_copy(
                emb_hbm.at[pl.ds(idx_ref[i], 1), :],
                emb_vmem.at[pl.ds(i, 1), :],
                dma_sem,
            ).start()
        for i in range(_CONTEXT):
            pltpu.make_async_copy(
                emb_hbm.at[pl.ds(idx_ref[i], 1), :],
                emb_vmem.at[pl.ds(i, 1), :],
                dma_sem,
            ).wait()
        acc = b1_ref[...]
        for i in range(_CONTEXT):
            acc = acc + jnp.dot(
                emb_vmem[pl.ds(i, 1), :],
                w1_ref[pl.ds(i * _EMBED, _EMBED), :],
                preferred_element_type=jnp.float32,
            )
        h_ref[...] = jnp.tanh(acc)

    @pl.when(p == 0)
    def _logits_and_stats():
        x = jnp.dot(h_ref[...], w2_ref[...], preferred_element_type=jnp.float32)
        x = x + b2_ref[...]
        col = j * _VB + jax.lax.broadcasted_iota(jnp.int32, (1, _VB), 1)
        x = jnp.where(col < _VOCAB, x, -jnp.inf)
        out_ref[...] = x
        bm = jnp.max(x)

        @pl.when(j == 0)
        def _():
            stat_ref[0] = bm
            stat_ref[1] = jnp.sum(jnp.exp(x - bm))

        @pl.when(j > 0)
        def _():
            m_old = stat_ref[0]
            m_new = jnp.maximum(m_old, bm)
            stat_ref[1] = stat_ref[1] * jnp.exp(m_old - m_new) + jnp.sum(
                jnp.exp(x - m_new)
            )
            stat_ref[0] = m_new

    @pl.when(p == 1)
    def _normalize():
        lse = stat_ref[0] + jnp.log(stat_ref[1])
        out_ref[...] = logits_in_ref[...] - lse


def kernel(inputs, emb_table, W1, b1, W2, b2):
    b1_2d = b1.reshape(1, _HIDDEN)
    b2_2d = b2.reshape(1, _VOCAB)
    logits_buf = jnp.zeros((1, _VOCAB), jnp.float32)

    out = pl.pallas_call(
        _body,
        grid_spec=pltpu.PrefetchScalarGridSpec(
            num_scalar_prefetch=1,
            grid=(2, _NVB),
            in_specs=[
                pl.BlockSpec(memory_space=pl.ANY),
                pl.BlockSpec((_CONTEXT * _EMBED, _HIDDEN), lambda p, j, idx: (0, 0)),
                pl.BlockSpec((1, _HIDDEN), lambda p, j, idx: (0, 0)),
                pl.BlockSpec(
                    (_HIDDEN, _VB),
                    lambda p, j, idx: (0, jax.lax.select(p == 0, j, 0)),
                ),
                pl.BlockSpec(
                    (1, _VB),
                    lambda p, j, idx: (0, jax.lax.select(p == 0, j, 0)),
                ),
                pl.BlockSpec((1, _VB), lambda p, j, idx: (0, j)),
            ],
            out_specs=pl.BlockSpec((1, _VB), lambda p, j, idx: (0, j)),
            scratch_shapes=[
                pltpu.VMEM((_CONTEXT, _EMBED), jnp.float32),
                pltpu.VMEM((1, _HIDDEN), jnp.float32),
                pltpu.SMEM((2,), jnp.float32),
                pltpu.SemaphoreType.DMA,
            ],
        ),
        out_shape=jax.ShapeDtypeStruct((1, _VOCAB), jnp.float32),
        input_output_aliases={6: 0},
        compiler_params=pltpu.CompilerParams(
            dimension_semantics=("arbitrary", "arbitrary"),
        ),
    )(inputs.astype(jnp.int32), emb_table, W1, b1_2d, W2, b2_2d, logits_buf)

    return out


# speedup vs baseline: 1.3712x; 1.0623x over previous
import jax
import jax.numpy as jnp
from jax.experimental import pallas as pl
from jax.experimental.pallas import tpu as pltpu

_CONTEXT = 20
_VOCAB = 100000
_EMBED = 64
_HIDDEN = 100

_VB = 16384
_NVB = (_VOCAB + _VB - 1) // _VB


def _body(
    idx_ref,
    emb_hbm,
    w1_ref,
    b1_ref,
    w2_ref,
    b2_ref,
    logits_in_ref,
    out_ref,
    emb_vmem,
    h_ref,
    stat_ref,
    dma_sem,
):
    p = pl.program_id(0)
    j = pl.program_id(1)

    @pl.when(jnp.logical_and(p == 0, j == 0))
    def _gather_and_hidden():
        for i in range(_CONTEXT):
            pltpu.make_async_copy(
                emb_hbm.at[pl.ds(idx_ref[i], 1), :],
                emb_vmem.at[pl.ds(i, 1), :],
                dma_sem,
            ).start()
        for i in range(_CONTEXT):
            pltpu.make_async_copy(
                emb_hbm.at[pl.ds(idx_ref[i], 1), :],
                emb_vmem.at[pl.ds(i, 1), :],
                dma_sem,
            ).wait()
        acc = b1_ref[...]
        for i in range(_CONTEXT):
            acc = acc + jnp.dot(
                emb_vmem[pl.ds(i, 1), :],
                w1_ref[pl.ds(i * _EMBED, _EMBED), :],
                preferred_element_type=jnp.float32,
            )
        h_ref[...] = jnp.tanh(acc)

    @pl.when(p == 0)
    def _logits_and_stats():
        x = jnp.dot(h_ref[...], w2_ref[...], preferred_element_type=jnp.float32)
        x = x + b2_ref[...]
        col = j * _VB + jax.lax.broadcasted_iota(jnp.int32, (1, _VB), 1)
        x = jnp.where(col < _VOCAB, x, -jnp.inf)
        out_ref[...] = x
        bm = jnp.max(x)

        @pl.when(j == 0)
        def _():
            stat_ref[0] = bm
            stat_ref[1] = jnp.sum(jnp.exp(x - bm))

        @pl.when(j > 0)
        def _():
            m_old = stat_ref[0]
            m_new = jnp.maximum(m_old, bm)
            stat_ref[1] = stat_ref[1] * jnp.exp(m_old - m_new) + jnp.sum(
                jnp.exp(x - m_new)
            )
            stat_ref[0] = m_new

    @pl.when(p == 1)
    def _normalize():
        lse = stat_ref[0] + jnp.log(stat_ref[1])
        out_ref[...] = logits_in_ref[...] - lse


def kernel(inputs, emb_table, W1, b1, W2, b2):
    b1_2d = b1.reshape(1, _HIDDEN)
    b2_2d = b2.reshape(1, _VOCAB)
    logits_buf = jnp.zeros((1, _VOCAB), jnp.float32)

    out = pl.pallas_call(
        _body,
        grid_spec=pltpu.PrefetchScalarGridSpec(
            num_scalar_prefetch=1,
            grid=(2, _NVB),
            in_specs=[
                pl.BlockSpec(memory_space=pl.ANY),
                pl.BlockSpec((_CONTEXT * _EMBED, _HIDDEN), lambda p, j, idx: (0, 0)),
                pl.BlockSpec((1, _HIDDEN), lambda p, j, idx: (0, 0)),
                pl.BlockSpec(
                    (_HIDDEN, _VB),
                    lambda p, j, idx: (0, jax.lax.select(p == 0, j, 0)),
                ),
                pl.BlockSpec(
                    (1, _VB),
                    lambda p, j, idx: (0, jax.lax.select(p == 0, j, 0)),
                ),
                pl.BlockSpec((1, _VB), lambda p, j, idx: (0, j)),
            ],
            out_specs=pl.BlockSpec((1, _VB), lambda p, j, idx: (0, j)),
            scratch_shapes=[
                pltpu.VMEM((_CONTEXT, _EMBED), jnp.float32),
                pltpu.VMEM((1, _HIDDEN), jnp.float32),
                pltpu.SMEM((2,), jnp.float32),
                pltpu.SemaphoreType.DMA,
            ],
        ),
        out_shape=jax.ShapeDtypeStruct((1, _VOCAB), jnp.float32),
        input_output_aliases={6: 0},
        compiler_params=pltpu.CompilerParams(
            dimension_semantics=("arbitrary", "arbitrary"),
        ),
    )(inputs.astype(jnp.int32), emb_table, W1, b1_2d, W2, b2_2d, logits_buf)

    return out
